# Initial kernel scaffold; baseline (speedup 1.0000x reference)
import jax
import jax.numpy as jnp
from jax.experimental import pallas as pl


def kernel(inputs, pids, vids):
    raise NotImplementedError("write your pallas kernel here")


# speedup vs baseline: 3.0915x; 3.0915x over previous
import functools

import jax
import jax.numpy as jnp
from jax import lax
from jax.experimental import pallas as pl
from jax.experimental.pallas import tpu as pltpu

_EPS = 0.1
_ALPHA = 0.2
_ROWS_PER_BLOCK = 8


def _loss_body(x_ref, pid_ref, vid_ref, out_ref):
    i = pl.program_id(0)
    n = pl.num_programs(0)
    x = x_ref[...]
    C = x.shape[1]
    B_total = n * x.shape[0]
    p3 = pid_ref[...] * 3
    vid = vid_ref[...]

    m = jnp.max(x, axis=1, keepdims=True)
    e = jnp.exp(x - m)
    z = jnp.sum(e, axis=1, keepdims=True)
    s = jnp.sum(x, axis=1, keepdims=True)

    col = lax.broadcasted_iota(jnp.int32, x.shape, 1)
    in3 = (col >= p3) & (col < p3 + 3)
    tmask = col == p3 + vid
    zero = jnp.zeros_like(x)
    s3 = jnp.sum(jnp.where(in3, x, zero), axis=1, keepdims=True)
    z3 = jnp.sum(jnp.where(in3, e, zero), axis=1, keepdims=True)
    st = jnp.sum(jnp.where(tmask, x, zero), axis=1, keepdims=True)
    zt = jnp.sum(jnp.where(tmask, e, zero), axis=1, keepdims=True)

    c = m + jnp.log(z)
    L = s - C * c
    L3 = s3 - 3.0 * c
    lpt = st - c
    e1 = _ALPHA * (1.0 - z3 / z)
    e2 = _ALPHA * (1.0 - zt / z)
    S = (e1 / (C - 3)) * (L - L3) + 0.5 * e2 * (L3 - lpt) + (1.0 - e1 - e2) * lpt
    contrib = (1.0 - _EPS) * S + (_EPS / C) * L
    bs = jnp.sum(contrib, axis=0, keepdims=True)

    @pl.when(i == 0)
    def _():
        out_ref[...] = jnp.zeros_like(out_ref)

    out_ref[...] += bs

    @pl.when(i == n - 1)
    def _():
        out_ref[...] = out_ref[...] * (-1.0 / B_total)


@jax.jit
def kernel(inputs, pids, vids):
    B, C = inputs.shape
    R = _ROWS_PER_BLOCK
    grid = B // R
    pids2 = pids.reshape(B, 1).astype(jnp.int32)
    vids2 = vids.reshape(B, 1).astype(jnp.int32)
    out = pl.pallas_call(
        _loss_body,
        grid=(grid,),
        in_specs=[
            pl.BlockSpec((R, C), lambda i: (i, 0)),
            pl.BlockSpec((R, 1), lambda i: (i, 0)),
            pl.BlockSpec((R, 1), lambda i: (i, 0)),
        ],
        out_specs=pl.BlockSpec((1, 1), lambda i: (0, 0)),
        out_shape=jax.ShapeDtypeStruct((1, 1), jnp.float32),
    )(inputs, pids2, vids2)
    return out[0, 0]
